# Initial kernel scaffold; baseline (speedup 1.0000x reference)
import jax
import jax.numpy as jnp
from jax.experimental import pallas as pl


def kernel(x, edge_index, W1, b1, W2, b2, Wf1, bf1, Wf3, bf3):
    raise NotImplementedError("write your pallas kernel here")


# speedup vs baseline: 4.6387x; 4.6387x over previous
import functools

import jax
import jax.numpy as jnp
from jax import lax
from jax.experimental import pallas as pl
from jax.experimental.pallas import tpu as pltpu
from jax.experimental.pallas import tpu_sc as plsc

N_NODES = 10000
N_EDGES = 320000
D = 128

NC = 2
NS = 16
NW = NC * NS
EDGES_PER_TILE = N_EDGES // NW
CHUNK = 80
CHUNKS_PER_TILE = EDGES_PER_TILE // CHUNK
ROWS_PER_TILE = N_NODES // NS

_sc_mesh = plsc.VectorSubcoreMesh(core_axis_name="c", subcore_axis_name="s")


N_PAD = 10240
ELEMS_PER_TILE = N_PAD // NS


@functools.partial(
    pl.kernel,
    out_type=jax.ShapeDtypeStruct((NC * 2 * N_PAD,), jnp.float32),
    mesh=_sc_mesh,
    scratch_types=[
        pltpu.VMEM((CHUNK,), jnp.int32),
        pltpu.VMEM((CHUNK,), jnp.int32),
        pltpu.VMEM((CHUNK,), jnp.float32),
        pltpu.VMEM_SHARED((N_PAD,), jnp.float32),
        pltpu.VMEM_SHARED((N_PAD,), jnp.float32),
    ],
)
def _sc_degrees(src_hbm, dst_hbm, zeros_hbm, out_hbm,
                idx_s, idx_d, ones_v, dego_sh, degi_sh):
    c = lax.axis_index("c")
    s = lax.axis_index("s")

    ones16 = jnp.ones((16,), jnp.float32)
    for j in range(CHUNK // 16):
        ones_v[pl.ds(16 * j, 16)] = ones16

    pltpu.sync_copy(zeros_hbm, dego_sh.at[pl.ds(s * ELEMS_PER_TILE, ELEMS_PER_TILE)])
    pltpu.sync_copy(zeros_hbm, degi_sh.at[pl.ds(s * ELEMS_PER_TILE, ELEMS_PER_TILE)])

    plsc.subcore_barrier()

    tile_base = (c * NS + s) * EDGES_PER_TILE

    def body(i, carry):
        base = tile_base + i * CHUNK
        pltpu.sync_copy(src_hbm.at[pl.ds(base, CHUNK)], idx_s)
        pltpu.sync_copy(dst_hbm.at[pl.ds(base, CHUNK)], idx_d)
        pltpu.sync_copy(ones_v, dego_sh.at[idx_s], add=True)
        pltpu.sync_copy(ones_v, degi_sh.at[idx_d], add=True)
        return carry

    lax.fori_loop(0, CHUNKS_PER_TILE, body, 0)

    plsc.subcore_barrier()

    off = s * ELEMS_PER_TILE
    pltpu.sync_copy(dego_sh.at[pl.ds(off, ELEMS_PER_TILE)],
                    out_hbm.at[pl.ds((c * 2 + 0) * N_PAD + off, ELEMS_PER_TILE)])
    pltpu.sync_copy(degi_sh.at[pl.ds(off, ELEMS_PER_TILE)],
                    out_hbm.at[pl.ds((c * 2 + 1) * N_PAD + off, ELEMS_PER_TILE)])


@functools.partial(
    pl.kernel,
    out_type=jax.ShapeDtypeStruct((NC, N_NODES, D), jnp.float32),
    mesh=_sc_mesh,
    scratch_types=[
        pltpu.VMEM((CHUNK,), jnp.int32),
        pltpu.VMEM((CHUNK,), jnp.int32),
        pltpu.VMEM((CHUNK, D), jnp.float32),
        pltpu.VMEM_SHARED((N_NODES, D), jnp.float32),
        pltpu.SemaphoreType.DMA,
    ],
)
def _sc_spmm(h_hbm, src_hbm, dst_hbm, zrows_hbm, out_hbm,
             idx_s, idx_d, rows_v, agg_sh, sem):
    c = lax.axis_index("c")
    s = lax.axis_index("s")

    @pl.when(s < 10)
    def _zero():
        pltpu.sync_copy(zrows_hbm, agg_sh.at[pl.ds(s * 1000, 1000)])

    plsc.subcore_barrier()

    tile_base = (c * NS + s) * EDGES_PER_TILE

    def body(i, carry):
        base = tile_base + i * CHUNK
        pltpu.sync_copy(src_hbm.at[pl.ds(base, CHUNK)], idx_s)
        pltpu.sync_copy(dst_hbm.at[pl.ds(base, CHUNK)], idx_d)
        pltpu.async_copy(h_hbm.at[idx_s], rows_v, sem).wait()
        pltpu.sync_copy(rows_v, agg_sh.at[idx_d], add=True)
        return carry

    lax.fori_loop(0, CHUNKS_PER_TILE, body, 0)

    plsc.subcore_barrier()

    @pl.when(s < 10)
    def _dump():
        pltpu.sync_copy(agg_sh.at[pl.ds(s * 1000, 1000)],
                        out_hbm.at[c, pl.ds(s * 1000, 1000)])


def _prep_body(x_ref, dp_ref, hs_ref, ro_ref, ri_ref):
    do = dp_ref[0, 0] + dp_ref[1, 0]
    di = dp_ref[0, 1] + dp_ref[1, 1]
    ro = lax.rsqrt(jnp.maximum(do, 1.0))
    ri = lax.rsqrt(jnp.maximum(di, 1.0))
    ro_ref[...] = ro
    ri_ref[...] = ri
    hs_ref[...] = x_ref[...] * ro


def _tc_prep(x, degp):
    return pl.pallas_call(
        _prep_body,
        out_shape=[
            jax.ShapeDtypeStruct((N_NODES, D), jnp.float32),
            jax.ShapeDtypeStruct((N_NODES, 1), jnp.float32),
            jax.ShapeDtypeStruct((N_NODES, 1), jnp.float32),
        ],
    )(x, degp)


ROW_BLK = 1000


def _layer_body(agg_ref, ri_ref, ro_ref, w_ref, b_ref, out_ref):
    a = (agg_ref[0] + agg_ref[1]) * ri_ref[...]
    y = jnp.dot(a, w_ref[...], preferred_element_type=jnp.float32) + b_ref[...]
    out_ref[...] = jnp.maximum(y, 0.0) * ro_ref[...]


def _tc_layer1(aggp, ri, ro, w, b):
    grid = (N_NODES // ROW_BLK,)
    return pl.pallas_call(
        _layer_body,
        grid=grid,
        in_specs=[
            pl.BlockSpec((NC, ROW_BLK, D), lambda i: (0, i, 0)),
            pl.BlockSpec((ROW_BLK, 1), lambda i: (i, 0)),
            pl.BlockSpec((ROW_BLK, 1), lambda i: (i, 0)),
            pl.BlockSpec((D, D), lambda i: (0, 0)),
            pl.BlockSpec((1, D), lambda i: (0, 0)),
        ],
        out_specs=pl.BlockSpec((ROW_BLK, D), lambda i: (i, 0)),
        out_shape=jax.ShapeDtypeStruct((N_NODES, D), jnp.float32),
    )(aggp, ri, ro, w, b)


def _final_body(agg_ref, ri_ref, w2_ref, b2_ref, wf1_ref, bf1_ref,
                wf3_ref, bf3_ref, out_ref):
    a = (agg_ref[0] + agg_ref[1]) * ri_ref[...]
    h2 = jnp.maximum(
        jnp.dot(a, w2_ref[...], preferred_element_type=jnp.float32) + b2_ref[...], 0.0)
    h3 = jnp.maximum(
        jnp.dot(h2, wf1_ref[...], preferred_element_type=jnp.float32) + bf1_ref[...], 0.0)
    out_ref[...] = (
        jnp.dot(h3, wf3_ref[...], preferred_element_type=jnp.float32) + bf3_ref[...])


def _tc_final(aggp, ri, w2, b2, wf1, bf1, wf3, bf3):
    grid = (N_NODES // ROW_BLK,)
    wspec = pl.BlockSpec((D, D), lambda i: (0, 0))
    bspec = pl.BlockSpec((1, D), lambda i: (0, 0))
    return pl.pallas_call(
        _final_body,
        grid=grid,
        in_specs=[
            pl.BlockSpec((NC, ROW_BLK, D), lambda i: (0, i, 0)),
            pl.BlockSpec((ROW_BLK, 1), lambda i: (i, 0)),
            wspec, bspec, wspec, bspec, wspec, bspec,
        ],
        out_specs=pl.BlockSpec((ROW_BLK, D), lambda i: (i, 0)),
        out_shape=jax.ShapeDtypeStruct((N_NODES, D), jnp.float32),
    )(aggp, ri, w2, b2, wf1, bf1, wf3, bf3)


def kernel(x, edge_index, W1, b1, W2, b2, Wf1, bf1, Wf3, bf3):
    src = edge_index[0].astype(jnp.int32)
    dst = edge_index[1].astype(jnp.int32)

    zeros1d = jnp.zeros((ELEMS_PER_TILE,), jnp.float32)
    zrows = jnp.zeros((1000, D), jnp.float32)

    degp = _sc_degrees(src, dst, zeros1d)
    degp4 = degp.reshape(NC, 2, N_PAD, 1)[:, :, :N_NODES, :]

    hs0, ro, ri = _tc_prep(x, degp4)

    agg1 = _sc_spmm(hs0, src, dst, zrows)
    h1s = _tc_layer1(agg1, ri, ro, W1, b1.reshape(1, D))

    agg2 = _sc_spmm(h1s, src, dst, zrows)
    z = _tc_final(agg2, ri, W2, b2.reshape(1, D),
                  Wf1, bf1.reshape(1, D), Wf3, bf3.reshape(1, D))
    return z


# speedup vs baseline: 12.5715x; 2.7101x over previous
import functools

import jax
import jax.numpy as jnp
from jax import lax
from jax.experimental import pallas as pl
from jax.experimental.pallas import tpu as pltpu
from jax.experimental.pallas import tpu_sc as plsc

N_NODES = 10000
N_EDGES = 320000
D = 128

NC = 2
NS = 16
NW = NC * NS
EDGES_PER_TILE = N_EDGES // NW
CHUNK = 80
CHUNKS_PER_TILE = EDGES_PER_TILE // CHUNK
ROWS_PER_TILE = N_NODES // NS

_sc_mesh = plsc.VectorSubcoreMesh(core_axis_name="c", subcore_axis_name="s")


N_PAD = 10240
ELEMS_PER_TILE = N_PAD // NS
DEG_NBUF = 2


def _fill_idx(dst_ref, src_ref, chunk_id):
    for j in range(CHUNK // 16):
        dst_ref[pl.ds(16 * j, 16)] = src_ref[pl.ds(chunk_id * CHUNK + 16 * j, 16)]


@functools.partial(
    pl.kernel,
    out_type=jax.ShapeDtypeStruct((NC * 2 * N_PAD,), jnp.float32),
    mesh=_sc_mesh,
    scratch_types=[
        pltpu.VMEM((EDGES_PER_TILE,), jnp.int32),
        pltpu.VMEM((EDGES_PER_TILE,), jnp.int32),
        pltpu.VMEM((DEG_NBUF, CHUNK), jnp.int32),
        pltpu.VMEM((DEG_NBUF, CHUNK), jnp.int32),
        pltpu.VMEM((CHUNK,), jnp.float32),
        pltpu.VMEM_SHARED((N_PAD,), jnp.float32),
        pltpu.VMEM_SHARED((N_PAD,), jnp.float32),
    ] + [pltpu.SemaphoreType.DMA] * (2 * DEG_NBUF),
)
def _sc_degrees(src_hbm, dst_hbm, zeros_hbm, out_hbm,
                idx_s_all, idx_d_all, sbuf, dbuf, ones_v, dego_sh, degi_sh,
                *sems):
    c = lax.axis_index("c")
    s = lax.axis_index("s")

    ones16 = jnp.ones((16,), jnp.float32)
    for j in range(CHUNK // 16):
        ones_v[pl.ds(16 * j, 16)] = ones16

    pltpu.sync_copy(zeros_hbm, dego_sh.at[pl.ds(s * ELEMS_PER_TILE, ELEMS_PER_TILE)])
    pltpu.sync_copy(zeros_hbm, degi_sh.at[pl.ds(s * ELEMS_PER_TILE, ELEMS_PER_TILE)])

    tile_base = (c * NS + s) * EDGES_PER_TILE
    pltpu.sync_copy(src_hbm.at[pl.ds(tile_base, EDGES_PER_TILE)], idx_s_all)
    pltpu.sync_copy(dst_hbm.at[pl.ds(tile_base, EDGES_PER_TILE)], idx_d_all)

    plsc.subcore_barrier()

    def issue(b, k):
        _fill_idx(sbuf.at[b], idx_s_all, k)
        _fill_idx(dbuf.at[b], idx_d_all, k)
        pltpu.async_copy(ones_v, dego_sh.at[sbuf.at[b]], sems[2 * b], add=True)
        pltpu.async_copy(ones_v, degi_sh.at[dbuf.at[b]], sems[2 * b + 1], add=True)

    def drain(b):
        pltpu.make_async_copy(ones_v, dego_sh.at[sbuf.at[b]], sems[2 * b]).wait()
        pltpu.make_async_copy(ones_v, degi_sh.at[dbuf.at[b]], sems[2 * b + 1]).wait()

    for b in range(DEG_NBUF):
        issue(b, b)

    upper = ((CHUNKS_PER_TILE + DEG_NBUF - 1) // DEG_NBUF) * DEG_NBUF

    @pl.loop(DEG_NBUF, upper, step=DEG_NBUF)
    def _main(g):
        for b in range(DEG_NBUF):
            k = g + b

            @pl.when(k < CHUNKS_PER_TILE)
            def _():
                drain(b)
                issue(b, k)

    for b in range(DEG_NBUF):
        drain(b)

    plsc.subcore_barrier()

    off = s * ELEMS_PER_TILE
    pltpu.sync_copy(dego_sh.at[pl.ds(off, ELEMS_PER_TILE)],
                    out_hbm.at[pl.ds((c * 2 + 0) * N_PAD + off, ELEMS_PER_TILE)])
    pltpu.sync_copy(degi_sh.at[pl.ds(off, ELEMS_PER_TILE)],
                    out_hbm.at[pl.ds((c * 2 + 1) * N_PAD + off, ELEMS_PER_TILE)])


SPMM_NBUF = 3


@functools.partial(
    pl.kernel,
    out_type=jax.ShapeDtypeStruct((NC, N_NODES, D), jnp.float32),
    mesh=_sc_mesh,
    scratch_types=[
        pltpu.VMEM((EDGES_PER_TILE,), jnp.int32),
        pltpu.VMEM((SPMM_NBUF, CHUNK), jnp.int32),
        pltpu.VMEM((SPMM_NBUF, CHUNK, D), jnp.float32),
        pltpu.VMEM_SHARED((N_NODES, D), jnp.float32),
    ] + [pltpu.SemaphoreType.DMA] * (2 * SPMM_NBUF),
)
def _sc_spmm(h_hbm, src_hbm, dst_hbm, zrows_hbm, out_hbm,
             idx_s_all, dbuf, rows, agg_sh, *sems):
    c = lax.axis_index("c")
    s = lax.axis_index("s")

    @pl.when(s < 10)
    def _zero():
        pltpu.sync_copy(zrows_hbm, agg_sh.at[pl.ds(s * 1000, 1000)])

    tile_base = (c * NS + s) * EDGES_PER_TILE
    pltpu.sync_copy(src_hbm.at[pl.ds(tile_base, EDGES_PER_TILE)], idx_s_all)

    plsc.subcore_barrier()

    def issue_gather(b, k):
        pltpu.async_copy(dst_hbm.at[pl.ds(tile_base + k * CHUNK, CHUNK)],
                         dbuf.at[b], sems[SPMM_NBUF + b])
        pltpu.async_copy(h_hbm.at[idx_s_all.at[pl.ds(k * CHUNK, CHUNK)]],
                         rows.at[b], sems[b])

    def wait_gather(b, k):
        pltpu.make_async_copy(dst_hbm.at[pl.ds(tile_base + k * CHUNK, CHUNK)],
                              dbuf.at[b], sems[SPMM_NBUF + b]).wait()
        pltpu.make_async_copy(h_hbm.at[idx_s_all.at[pl.ds(k * CHUNK, CHUNK)]],
                              rows.at[b], sems[b]).wait()

    for b in range(SPMM_NBUF):
        issue_gather(b, b)

    upper = ((CHUNKS_PER_TILE + SPMM_NBUF - 1) // SPMM_NBUF) * SPMM_NBUF

    @pl.loop(0, upper, step=SPMM_NBUF)
    def _main(g):
        for b in range(SPMM_NBUF):
            k = g + b

            @pl.when(k < CHUNKS_PER_TILE)
            def _():
                wait_gather(b, k)
                pltpu.sync_copy(rows.at[b], agg_sh.at[dbuf.at[b]], add=True)

                @pl.when(k + SPMM_NBUF < CHUNKS_PER_TILE)
                def _():
                    issue_gather(b, k + SPMM_NBUF)

    plsc.subcore_barrier()

    @pl.when(s < 10)
    def _dump():
        pltpu.sync_copy(agg_sh.at[pl.ds(s * 1000, 1000)],
                        out_hbm.at[c, pl.ds(s * 1000, 1000)])


def _prep_body(x_ref, dp_ref, hs_ref, ro_ref, ri_ref):
    do = dp_ref[0, 0] + dp_ref[1, 0]
    di = dp_ref[0, 1] + dp_ref[1, 1]
    ro = lax.rsqrt(jnp.maximum(do, 1.0))
    ri = lax.rsqrt(jnp.maximum(di, 1.0))
    ro_ref[...] = ro
    ri_ref[...] = ri
    hs_ref[...] = x_ref[...] * ro


def _tc_prep(x, degp):
    return pl.pallas_call(
        _prep_body,
        out_shape=[
            jax.ShapeDtypeStruct((N_NODES, D), jnp.float32),
            jax.ShapeDtypeStruct((N_NODES, 1), jnp.float32),
            jax.ShapeDtypeStruct((N_NODES, 1), jnp.float32),
        ],
    )(x, degp)


ROW_BLK = 1000


def _layer_body(agg_ref, ri_ref, ro_ref, w_ref, b_ref, out_ref):
    a = (agg_ref[0] + agg_ref[1]) * ri_ref[...]
    y = jnp.dot(a, w_ref[...], preferred_element_type=jnp.float32) + b_ref[...]
    out_ref[...] = jnp.maximum(y, 0.0) * ro_ref[...]


def _tc_layer1(aggp, ri, ro, w, b):
    grid = (N_NODES // ROW_BLK,)
    return pl.pallas_call(
        _layer_body,
        grid=grid,
        in_specs=[
            pl.BlockSpec((NC, ROW_BLK, D), lambda i: (0, i, 0)),
            pl.BlockSpec((ROW_BLK, 1), lambda i: (i, 0)),
            pl.BlockSpec((ROW_BLK, 1), lambda i: (i, 0)),
            pl.BlockSpec((D, D), lambda i: (0, 0)),
            pl.BlockSpec((1, D), lambda i: (0, 0)),
        ],
        out_specs=pl.BlockSpec((ROW_BLK, D), lambda i: (i, 0)),
        out_shape=jax.ShapeDtypeStruct((N_NODES, D), jnp.float32),
    )(aggp, ri, ro, w, b)


def _final_body(agg_ref, ri_ref, w2_ref, b2_ref, wf1_ref, bf1_ref,
                wf3_ref, bf3_ref, out_ref):
    a = (agg_ref[0] + agg_ref[1]) * ri_ref[...]
    h2 = jnp.maximum(
        jnp.dot(a, w2_ref[...], preferred_element_type=jnp.float32) + b2_ref[...], 0.0)
    h3 = jnp.maximum(
        jnp.dot(h2, wf1_ref[...], preferred_element_type=jnp.float32) + bf1_ref[...], 0.0)
    out_ref[...] = (
        jnp.dot(h3, wf3_ref[...], preferred_element_type=jnp.float32) + bf3_ref[...])


def _tc_final(aggp, ri, w2, b2, wf1, bf1, wf3, bf3):
    grid = (N_NODES // ROW_BLK,)
    wspec = pl.BlockSpec((D, D), lambda i: (0, 0))
    bspec = pl.BlockSpec((1, D), lambda i: (0, 0))
    return pl.pallas_call(
        _final_body,
        grid=grid,
        in_specs=[
            pl.BlockSpec((NC, ROW_BLK, D), lambda i: (0, i, 0)),
            pl.BlockSpec((ROW_BLK, 1), lambda i: (i, 0)),
            wspec, bspec, wspec, bspec, wspec, bspec,
        ],
        out_specs=pl.BlockSpec((ROW_BLK, D), lambda i: (i, 0)),
        out_shape=jax.ShapeDtypeStruct((N_NODES, D), jnp.float32),
    )(aggp, ri, w2, b2, wf1, bf1, wf3, bf3)


def kernel(x, edge_index, W1, b1, W2, b2, Wf1, bf1, Wf3, bf3):
    src = edge_index[0].astype(jnp.int32)
    dst = edge_index[1].astype(jnp.int32)

    zeros1d = jnp.zeros((ELEMS_PER_TILE,), jnp.float32)
    zrows = jnp.zeros((1000, D), jnp.float32)

    degp = _sc_degrees(src, dst, zeros1d)
    degp4 = degp.reshape(NC, 2, N_PAD, 1)[:, :, :N_NODES, :]

    hs0, ro, ri = _tc_prep(x, degp4)

    agg1 = _sc_spmm(hs0, src, dst, zrows)
    h1s = _tc_layer1(agg1, ri, ro, W1, b1.reshape(1, D))

    agg2 = _sc_spmm(h1s, src, dst, zrows)
    z = _tc_final(agg2, ri, W2, b2.reshape(1, D),
                  Wf1, bf1.reshape(1, D), Wf3, bf3.reshape(1, D))
    return z


# speedup vs baseline: 14.4563x; 1.1499x over previous
import functools

import jax
import jax.numpy as jnp
from jax import lax
from jax.experimental import pallas as pl
from jax.experimental.pallas import tpu as pltpu
from jax.experimental.pallas import tpu_sc as plsc

N_NODES = 10000
N_EDGES = 320000
D = 128

NC = 2
NS = 16
NW = NC * NS
EDGES_PER_TILE = N_EDGES // NW
CHUNK = 80
CHUNKS_PER_TILE = EDGES_PER_TILE // CHUNK
ROWS_PER_TILE = N_NODES // NS

_sc_mesh = plsc.VectorSubcoreMesh(core_axis_name="c", subcore_axis_name="s")


N_PAD = 10240
ELEMS_PER_TILE = N_PAD // NS
DEG_NBUF = 2


def _fill_idx(dst_ref, src_ref, chunk_id):
    for j in range(CHUNK // 16):
        dst_ref[pl.ds(16 * j, 16)] = src_ref[pl.ds(chunk_id * CHUNK + 16 * j, 16)]


@functools.partial(
    pl.kernel,
    out_type=jax.ShapeDtypeStruct((NC * 2 * N_PAD,), jnp.float32),
    mesh=_sc_mesh,
    scratch_types=[
        pltpu.VMEM((EDGES_PER_TILE,), jnp.int32),
        pltpu.VMEM((EDGES_PER_TILE,), jnp.int32),
        pltpu.VMEM((DEG_NBUF, CHUNK), jnp.int32),
        pltpu.VMEM((DEG_NBUF, CHUNK), jnp.int32),
        pltpu.VMEM((CHUNK,), jnp.float32),
        pltpu.VMEM_SHARED((N_PAD,), jnp.float32),
        pltpu.VMEM_SHARED((N_PAD,), jnp.float32),
    ] + [pltpu.SemaphoreType.DMA] * (2 * DEG_NBUF),
)
def _sc_degrees(src_hbm, dst_hbm, zeros_hbm, out_hbm,
                idx_s_all, idx_d_all, sbuf, dbuf, ones_v, dego_sh, degi_sh,
                *sems):
    c = lax.axis_index("c")
    s = lax.axis_index("s")

    ones16 = jnp.ones((16,), jnp.float32)
    for j in range(CHUNK // 16):
        ones_v[pl.ds(16 * j, 16)] = ones16

    pltpu.sync_copy(zeros_hbm, dego_sh.at[pl.ds(s * ELEMS_PER_TILE, ELEMS_PER_TILE)])
    pltpu.sync_copy(zeros_hbm, degi_sh.at[pl.ds(s * ELEMS_PER_TILE, ELEMS_PER_TILE)])

    tile_base = (c * NS + s) * EDGES_PER_TILE
    pltpu.sync_copy(src_hbm.at[pl.ds(tile_base, EDGES_PER_TILE)], idx_s_all)
    pltpu.sync_copy(dst_hbm.at[pl.ds(tile_base, EDGES_PER_TILE)], idx_d_all)

    plsc.subcore_barrier()

    def issue(b, k):
        _fill_idx(sbuf.at[b], idx_s_all, k)
        _fill_idx(dbuf.at[b], idx_d_all, k)
        pltpu.async_copy(ones_v, dego_sh.at[sbuf.at[b]], sems[2 * b], add=True)
        pltpu.async_copy(ones_v, degi_sh.at[dbuf.at[b]], sems[2 * b + 1], add=True)

    def drain(b):
        pltpu.make_async_copy(ones_v, dego_sh.at[sbuf.at[b]], sems[2 * b]).wait()
        pltpu.make_async_copy(ones_v, degi_sh.at[dbuf.at[b]], sems[2 * b + 1]).wait()

    for b in range(DEG_NBUF):
        issue(b, b)

    upper = ((CHUNKS_PER_TILE + DEG_NBUF - 1) // DEG_NBUF) * DEG_NBUF

    @pl.loop(DEG_NBUF, upper, step=DEG_NBUF)
    def _main(g):
        for b in range(DEG_NBUF):
            k = g + b

            @pl.when(k < CHUNKS_PER_TILE)
            def _():
                drain(b)
                issue(b, k)

    for b in range(DEG_NBUF):
        drain(b)

    plsc.subcore_barrier()

    off = s * ELEMS_PER_TILE
    pltpu.sync_copy(dego_sh.at[pl.ds(off, ELEMS_PER_TILE)],
                    out_hbm.at[pl.ds((c * 2 + 0) * N_PAD + off, ELEMS_PER_TILE)])
    pltpu.sync_copy(degi_sh.at[pl.ds(off, ELEMS_PER_TILE)],
                    out_hbm.at[pl.ds((c * 2 + 1) * N_PAD + off, ELEMS_PER_TILE)])


SPMM_NBUF = 3


@functools.partial(
    pl.kernel,
    out_type=jax.ShapeDtypeStruct((NC, N_NODES, D), jnp.float32),
    mesh=_sc_mesh,
    scratch_types=[
        pltpu.VMEM((EDGES_PER_TILE,), jnp.int32),
        pltpu.VMEM((SPMM_NBUF, CHUNK), jnp.int32),
        pltpu.VMEM((SPMM_NBUF, CHUNK, D), jnp.float32),
        pltpu.VMEM_SHARED((N_NODES, D), jnp.float32),
    ] + [pltpu.SemaphoreType.DMA] * (2 * SPMM_NBUF),
)
def _sc_spmm(h_hbm, src_hbm, dst_hbm, zrows_hbm, out_hbm,
             idx_s_all, dbuf, rows, agg_sh, *sems):
    c = lax.axis_index("c")
    s = lax.axis_index("s")

    @pl.when(s < 10)
    def _zero():
        pltpu.sync_copy(zrows_hbm, agg_sh.at[pl.ds(s * 1000, 1000)])

    tile_base = (c * NS + s) * EDGES_PER_TILE
    pltpu.sync_copy(src_hbm.at[pl.ds(tile_base, EDGES_PER_TILE)], idx_s_all)

    plsc.subcore_barrier()

    def issue_gather(b, k):
        pltpu.async_copy(dst_hbm.at[pl.ds(tile_base + k * CHUNK, CHUNK)],
                         dbuf.at[b], sems[SPMM_NBUF + b])
        pltpu.async_copy(h_hbm.at[idx_s_all.at[pl.ds(k * CHUNK, CHUNK)]],
                         rows.at[b], sems[b])

    def wait_gather(b, k):
        pltpu.make_async_copy(dst_hbm.at[pl.ds(tile_base + k * CHUNK, CHUNK)],
                              dbuf.at[b], sems[SPMM_NBUF + b]).wait()
        pltpu.make_async_copy(h_hbm.at[idx_s_all.at[pl.ds(k * CHUNK, CHUNK)]],
                              rows.at[b], sems[b]).wait()

    for b in range(SPMM_NBUF):
        issue_gather(b, b)

    upper = ((CHUNKS_PER_TILE + SPMM_NBUF - 1) // SPMM_NBUF) * SPMM_NBUF

    @pl.loop(0, upper, step=SPMM_NBUF)
    def _main(g):
        for b in range(SPMM_NBUF):
            k = g + b

            @pl.when(k < CHUNKS_PER_TILE)
            def _():
                wait_gather(b, k)
                pltpu.sync_copy(rows.at[b], agg_sh.at[dbuf.at[b]], add=True)

                @pl.when(k + SPMM_NBUF < CHUNKS_PER_TILE)
                def _():
                    issue_gather(b, k + SPMM_NBUF)

    plsc.subcore_barrier()

    @pl.when(s < 10)
    def _dump():
        pltpu.sync_copy(agg_sh.at[pl.ds(s * 1000, 1000)],
                        out_hbm.at[c, pl.ds(s * 1000, 1000)])


def _prep_body(x_ref, dp_ref, hs_ref, ro_ref, ri_ref):
    do = dp_ref[pl.ds(0, N_PAD)] + dp_ref[pl.ds(2 * N_PAD, N_PAD)]
    di = dp_ref[pl.ds(N_PAD, N_PAD)] + dp_ref[pl.ds(3 * N_PAD, N_PAD)]
    ro = lax.rsqrt(jnp.maximum(do, 1.0)).reshape(N_PAD, 1)[:N_NODES]
    ri = lax.rsqrt(jnp.maximum(di, 1.0)).reshape(N_PAD, 1)[:N_NODES]
    ro_ref[...] = ro
    ri_ref[...] = ri
    hs_ref[...] = x_ref[...] * ro


def _tc_prep(x, degp):
    return pl.pallas_call(
        _prep_body,
        out_shape=[
            jax.ShapeDtypeStruct((N_NODES, D), jnp.float32),
            jax.ShapeDtypeStruct((N_NODES, 1), jnp.float32),
            jax.ShapeDtypeStruct((N_NODES, 1), jnp.float32),
        ],
    )(x, degp)


ROW_BLK = 1000


def _layer_body(agg_ref, ri_ref, ro_ref, w_ref, b_ref, out_ref):
    a = (agg_ref[0] + agg_ref[1]) * ri_ref[...]
    y = jnp.dot(a, w_ref[...], preferred_element_type=jnp.float32) + b_ref[...]
    out_ref[...] = jnp.maximum(y, 0.0) * ro_ref[...]


def _tc_layer1(aggp, ri, ro, w, b):
    grid = (N_NODES // ROW_BLK,)
    return pl.pallas_call(
        _layer_body,
        grid=grid,
        in_specs=[
            pl.BlockSpec((NC, ROW_BLK, D), lambda i: (0, i, 0)),
            pl.BlockSpec((ROW_BLK, 1), lambda i: (i, 0)),
            pl.BlockSpec((ROW_BLK, 1), lambda i: (i, 0)),
            pl.BlockSpec((D, D), lambda i: (0, 0)),
            pl.BlockSpec((1, D), lambda i: (0, 0)),
        ],
        out_specs=pl.BlockSpec((ROW_BLK, D), lambda i: (i, 0)),
        out_shape=jax.ShapeDtypeStruct((N_NODES, D), jnp.float32),
    )(aggp, ri, ro, w, b)


def _final_body(agg_ref, ri_ref, w2_ref, b2_ref, wf1_ref, bf1_ref,
                wf3_ref, bf3_ref, out_ref):
    a = (agg_ref[0] + agg_ref[1]) * ri_ref[...]
    h2 = jnp.maximum(
        jnp.dot(a, w2_ref[...], preferred_element_type=jnp.float32) + b2_ref[...], 0.0)
    h3 = jnp.maximum(
        jnp.dot(h2, wf1_ref[...], preferred_element_type=jnp.float32) + bf1_ref[...], 0.0)
    out_ref[...] = (
        jnp.dot(h3, wf3_ref[...], preferred_element_type=jnp.float32) + bf3_ref[...])


def _tc_final(aggp, ri, w2, b2, wf1, bf1, wf3, bf3):
    grid = (N_NODES // ROW_BLK,)
    wspec = pl.BlockSpec((D, D), lambda i: (0, 0))
    bspec = pl.BlockSpec((1, D), lambda i: (0, 0))
    return pl.pallas_call(
        _final_body,
        grid=grid,
        in_specs=[
            pl.BlockSpec((NC, ROW_BLK, D), lambda i: (0, i, 0)),
            pl.BlockSpec((ROW_BLK, 1), lambda i: (i, 0)),
            wspec, bspec, wspec, bspec, wspec, bspec,
        ],
        out_specs=pl.BlockSpec((ROW_BLK, D), lambda i: (i, 0)),
        out_shape=jax.ShapeDtypeStruct((N_NODES, D), jnp.float32),
    )(aggp, ri, w2, b2, wf1, bf1, wf3, bf3)


def kernel(x, edge_index, W1, b1, W2, b2, Wf1, bf1, Wf3, bf3):
    src = edge_index[0].astype(jnp.int32)
    dst = edge_index[1].astype(jnp.int32)

    zeros1d = jnp.zeros((ELEMS_PER_TILE,), jnp.float32)
    zrows = jnp.zeros((1000, D), jnp.float32)

    degp = _sc_degrees(src, dst, zeros1d)

    hs0, ro, ri = _tc_prep(x, degp)

    agg1 = _sc_spmm(hs0, src, dst, zrows)
    h1s = _tc_layer1(agg1, ri, ro, W1, b1.reshape(1, D))

    agg2 = _sc_spmm(h1s, src, dst, zrows)
    z = _tc_final(agg2, ri, W2, b2.reshape(1, D),
                  Wf1, bf1.reshape(1, D), Wf3, bf3.reshape(1, D))
    return z


# speedup vs baseline: 14.9567x; 1.0346x over previous
import functools

import jax
import jax.numpy as jnp
from jax import lax
from jax.experimental import pallas as pl
from jax.experimental.pallas import tpu as pltpu
from jax.experimental.pallas import tpu_sc as plsc

N_NODES = 10000
N_EDGES = 320000
D = 128

NC = 2
NS = 16
NW = NC * NS
EDGES_PER_TILE = N_EDGES // NW
CHUNK = 80
CHUNKS_PER_TILE = EDGES_PER_TILE // CHUNK
ROWS_PER_TILE = N_NODES // NS

_sc_mesh = plsc.VectorSubcoreMesh(core_axis_name="c", subcore_axis_name="s")


N_PAD = 10240
ELEMS_PER_TILE = N_PAD // NS
ECHUNK = 128
TOT_ECHUNKS = N_EDGES // ECHUNK
DEG_RING = 4


@functools.partial(
    pl.kernel,
    out_type=[
        jax.ShapeDtypeStruct((NC * 2 * N_PAD,), jnp.float32),
        jax.ShapeDtypeStruct((N_EDGES,), jnp.int32),
        jax.ShapeDtypeStruct((N_EDGES,), jnp.int32),
    ],
    mesh=_sc_mesh,
    scratch_types=[
        pltpu.VMEM((DEG_RING, 2, ECHUNK), jnp.int32),
        pltpu.VMEM((ECHUNK,), jnp.float32),
        pltpu.VMEM_SHARED((N_PAD,), jnp.float32),
        pltpu.VMEM_SHARED((N_PAD,), jnp.float32),
    ] + [pltpu.SemaphoreType.DMA] * (3 * DEG_RING),
)
def _sc_degrees(edge_hbm, zeros_hbm, out_hbm, src_out, dst_out,
                ebuf, ones_v, dego_sh, degi_sh, *sems):
    c = lax.axis_index("c")
    s = lax.axis_index("s")
    w = c * NS + s

    esem = sems[0:DEG_RING]
    ssem = sems[DEG_RING:2 * DEG_RING]
    osem = sems[2 * DEG_RING:3 * DEG_RING]

    ones16 = jnp.ones((16,), jnp.float32)
    for j in range(ECHUNK // 16):
        ones_v[pl.ds(16 * j, 16)] = ones16

    pltpu.sync_copy(zeros_hbm, dego_sh.at[pl.ds(s * ELEMS_PER_TILE, ELEMS_PER_TILE)])
    pltpu.sync_copy(zeros_hbm, degi_sh.at[pl.ds(s * ELEMS_PER_TILE, ELEMS_PER_TILE)])

    plsc.subcore_barrier()

    def chunk_of(j):
        return w + NW * j

    def issue_in(b, j):
        k = chunk_of(j)
        pltpu.async_copy(edge_hbm.at[:, pl.ds(k * ECHUNK, ECHUNK)], ebuf.at[b],
                         esem[b])

    def wait_in(b, j):
        k = chunk_of(j)
        pltpu.make_async_copy(edge_hbm.at[:, pl.ds(k * ECHUNK, ECHUNK)],
                              ebuf.at[b], esem[b]).wait()

    def issue_work(b, j):
        k = chunk_of(j)
        pltpu.async_copy(ones_v, dego_sh.at[ebuf.at[b, 0]], ssem[b], add=True)
        pltpu.async_copy(ones_v, degi_sh.at[ebuf.at[b, 1]], ssem[b], add=True)
        pltpu.async_copy(ebuf.at[b, 0], src_out.at[pl.ds(k * ECHUNK, ECHUNK)],
                         osem[b])
        pltpu.async_copy(ebuf.at[b, 1], dst_out.at[pl.ds(k * ECHUNK, ECHUNK)],
                         osem[b])

    def drain_work(b, j):
        k = chunk_of(j)
        pltpu.make_async_copy(ones_v, dego_sh.at[ebuf.at[b, 0]], ssem[b]).wait()
        pltpu.make_async_copy(ones_v, degi_sh.at[ebuf.at[b, 1]], ssem[b]).wait()
        pltpu.make_async_copy(ebuf.at[b, 0],
                              src_out.at[pl.ds(k * ECHUNK, ECHUNK)], osem[b]).wait()
        pltpu.make_async_copy(ebuf.at[b, 1],
                              dst_out.at[pl.ds(k * ECHUNK, ECHUNK)], osem[b]).wait()

    issue_in(0, 0)
    issue_in(1, 1)

    def valid(j):
        return chunk_of(j) < TOT_ECHUNKS

    @pl.loop(0, 80, step=DEG_RING)
    def _main(g):
        for bi in range(DEG_RING):
            j = g + bi
            b = bi
            b2 = (bi + 2) % DEG_RING

            @pl.when(jnp.logical_and(j >= 2, valid(j - 2)))
            def _():
                drain_work(b2, j - 2)

            @pl.when(valid(j + 2))
            def _():
                issue_in(b2, j + 2)

            @pl.when(valid(j))
            def _():
                wait_in(b, j)
                issue_work(b, j)

    @pl.when(valid(78))
    def _():
        drain_work(78 % DEG_RING, 78)

    plsc.subcore_barrier()

    off = s * ELEMS_PER_TILE
    pltpu.sync_copy(dego_sh.at[pl.ds(off, ELEMS_PER_TILE)],
                    out_hbm.at[pl.ds((c * 2 + 0) * N_PAD + off, ELEMS_PER_TILE)])
    pltpu.sync_copy(degi_sh.at[pl.ds(off, ELEMS_PER_TILE)],
                    out_hbm.at[pl.ds((c * 2 + 1) * N_PAD + off, ELEMS_PER_TILE)])


SPMM_NBUF = 3


@functools.partial(
    pl.kernel,
    out_type=jax.ShapeDtypeStruct((NC, N_NODES, D), jnp.float32),
    mesh=_sc_mesh,
    scratch_types=[
        pltpu.VMEM((EDGES_PER_TILE,), jnp.int32),
        pltpu.VMEM((SPMM_NBUF, CHUNK), jnp.int32),
        pltpu.VMEM((SPMM_NBUF, CHUNK, D), jnp.float32),
        pltpu.VMEM_SHARED((N_NODES, D), jnp.float32),
    ] + [pltpu.SemaphoreType.DMA] * (2 * SPMM_NBUF),
)
def _sc_spmm(h_hbm, src_hbm, dst_hbm, zrows_hbm, out_hbm,
             idx_s_all, dbuf, rows, agg_sh, *sems):
    c = lax.axis_index("c")
    s = lax.axis_index("s")

    @pl.when(s < 10)
    def _zero():
        pltpu.sync_copy(zrows_hbm, agg_sh.at[pl.ds(s * 1000, 1000)])

    tile_base = (c * NS + s) * EDGES_PER_TILE
    pltpu.sync_copy(src_hbm.at[pl.ds(tile_base, EDGES_PER_TILE)], idx_s_all)

    plsc.subcore_barrier()

    def issue_gather(b, k):
        pltpu.async_copy(dst_hbm.at[pl.ds(tile_base + k * CHUNK, CHUNK)],
                         dbuf.at[b], sems[SPMM_NBUF + b])
        pltpu.async_copy(h_hbm.at[idx_s_all.at[pl.ds(k * CHUNK, CHUNK)]],
                         rows.at[b], sems[b])

    def wait_gather(b, k):
        pltpu.make_async_copy(dst_hbm.at[pl.ds(tile_base + k * CHUNK, CHUNK)],
                              dbuf.at[b], sems[SPMM_NBUF + b]).wait()
        pltpu.make_async_copy(h_hbm.at[idx_s_all.at[pl.ds(k * CHUNK, CHUNK)]],
                              rows.at[b], sems[b]).wait()

    for b in range(SPMM_NBUF):
        issue_gather(b, b)

    upper = ((CHUNKS_PER_TILE + SPMM_NBUF - 1) // SPMM_NBUF) * SPMM_NBUF

    @pl.loop(0, upper, step=SPMM_NBUF)
    def _main(g):
        for b in range(SPMM_NBUF):
            k = g + b

            @pl.when(k < CHUNKS_PER_TILE)
            def _():
                wait_gather(b, k)
                pltpu.sync_copy(rows.at[b], agg_sh.at[dbuf.at[b]], add=True)

                @pl.when(k + SPMM_NBUF < CHUNKS_PER_TILE)
                def _():
                    issue_gather(b, k + SPMM_NBUF)

    plsc.subcore_barrier()

    @pl.when(s < 10)
    def _dump():
        pltpu.sync_copy(agg_sh.at[pl.ds(s * 1000, 1000)],
                        out_hbm.at[c, pl.ds(s * 1000, 1000)])


def _prep_body(x_ref, dp_ref, hs_ref, ro_ref, ri_ref):
    do = dp_ref[pl.ds(0, N_PAD)] + dp_ref[pl.ds(2 * N_PAD, N_PAD)]
    di = dp_ref[pl.ds(N_PAD, N_PAD)] + dp_ref[pl.ds(3 * N_PAD, N_PAD)]
    ro = lax.rsqrt(jnp.maximum(do, 1.0)).reshape(N_PAD, 1)[:N_NODES]
    ri = lax.rsqrt(jnp.maximum(di, 1.0)).reshape(N_PAD, 1)[:N_NODES]
    ro_ref[...] = ro
    ri_ref[...] = ri
    hs_ref[...] = x_ref[...] * ro


def _tc_prep(x, degp):
    return pl.pallas_call(
        _prep_body,
        out_shape=[
            jax.ShapeDtypeStruct((N_NODES, D), jnp.float32),
            jax.ShapeDtypeStruct((N_NODES, 1), jnp.float32),
            jax.ShapeDtypeStruct((N_NODES, 1), jnp.float32),
        ],
    )(x, degp)


ROW_BLK = 1000


def _layer_body(agg_ref, ri_ref, ro_ref, w_ref, b_ref, out_ref):
    a = (agg_ref[0] + agg_ref[1]) * ri_ref[...]
    y = jnp.dot(a, w_ref[...], preferred_element_type=jnp.float32) + b_ref[...]
    out_ref[...] = jnp.maximum(y, 0.0) * ro_ref[...]


def _tc_layer1(aggp, ri, ro, w, b):
    grid = (N_NODES // ROW_BLK,)
    return pl.pallas_call(
        _layer_body,
        grid=grid,
        in_specs=[
            pl.BlockSpec((NC, ROW_BLK, D), lambda i: (0, i, 0)),
            pl.BlockSpec((ROW_BLK, 1), lambda i: (i, 0)),
            pl.BlockSpec((ROW_BLK, 1), lambda i: (i, 0)),
            pl.BlockSpec((D, D), lambda i: (0, 0)),
            pl.BlockSpec((1, D), lambda i: (0, 0)),
        ],
        out_specs=pl.BlockSpec((ROW_BLK, D), lambda i: (i, 0)),
        out_shape=jax.ShapeDtypeStruct((N_NODES, D), jnp.float32),
    )(aggp, ri, ro, w, b)


def _final_body(agg_ref, ri_ref, w2_ref, b2_ref, wf1_ref, bf1_ref,
                wf3_ref, bf3_ref, out_ref):
    a = (agg_ref[0] + agg_ref[1]) * ri_ref[...]
    h2 = jnp.maximum(
        jnp.dot(a, w2_ref[...], preferred_element_type=jnp.float32) + b2_ref[...], 0.0)
    h3 = jnp.maximum(
        jnp.dot(h2, wf1_ref[...], preferred_element_type=jnp.float32) + bf1_ref[...], 0.0)
    out_ref[...] = (
        jnp.dot(h3, wf3_ref[...], preferred_element_type=jnp.float32) + bf3_ref[...])


def _tc_final(aggp, ri, w2, b2, wf1, bf1, wf3, bf3):
    grid = (N_NODES // ROW_BLK,)
    wspec = pl.BlockSpec((D, D), lambda i: (0, 0))
    bspec = pl.BlockSpec((1, D), lambda i: (0, 0))
    return pl.pallas_call(
        _final_body,
        grid=grid,
        in_specs=[
            pl.BlockSpec((NC, ROW_BLK, D), lambda i: (0, i, 0)),
            pl.BlockSpec((ROW_BLK, 1), lambda i: (i, 0)),
            wspec, bspec, wspec, bspec, wspec, bspec,
        ],
        out_specs=pl.BlockSpec((ROW_BLK, D), lambda i: (i, 0)),
        out_shape=jax.ShapeDtypeStruct((N_NODES, D), jnp.float32),
    )(aggp, ri, w2, b2, wf1, bf1, wf3, bf3)


def kernel(x, edge_index, W1, b1, W2, b2, Wf1, bf1, Wf3, bf3):
    ei = edge_index.astype(jnp.int32)

    zeros1d = jnp.zeros((ELEMS_PER_TILE,), jnp.float32)
    zrows = jnp.zeros((1000, D), jnp.float32)

    degp, src, dst = _sc_degrees(ei, zeros1d)

    hs0, ro, ri = _tc_prep(x, degp)

    agg1 = _sc_spmm(hs0, src, dst, zrows)
    h1s = _tc_layer1(agg1, ri, ro, W1, b1.reshape(1, D))

    agg2 = _sc_spmm(h1s, src, dst, zrows)
    z = _tc_final(agg2, ri, W2, b2.reshape(1, D),
                  Wf1, bf1.reshape(1, D), Wf3, bf3.reshape(1, D))
    return z


# speedup vs baseline: 15.2791x; 1.0216x over previous
import functools

import jax
import jax.numpy as jnp
from jax import lax
from jax.experimental import pallas as pl
from jax.experimental.pallas import tpu as pltpu
from jax.experimental.pallas import tpu_sc as plsc

N_NODES = 10000
N_EDGES = 320000
D = 128

NC = 2
NS = 16
NW = NC * NS
EDGES_PER_TILE = N_EDGES // NW
CHUNK = 80
CHUNKS_PER_TILE = EDGES_PER_TILE // CHUNK
ROWS_PER_TILE = N_NODES // NS

_sc_mesh = plsc.VectorSubcoreMesh(core_axis_name="c", subcore_axis_name="s")


N_PAD = 10240
ELEMS_PER_TILE = N_PAD // NS
ECHUNK = 128
TOT_ECHUNKS = N_EDGES // ECHUNK
DEG_RING = 4


@functools.partial(
    pl.kernel,
    out_type=jax.ShapeDtypeStruct((NC * 2 * N_PAD,), jnp.float32),
    mesh=_sc_mesh,
    scratch_types=[
        pltpu.VMEM((DEG_RING, 2, ECHUNK), jnp.int32),
        pltpu.VMEM((ECHUNK,), jnp.float32),
        pltpu.VMEM_SHARED((N_PAD,), jnp.float32),
        pltpu.VMEM_SHARED((N_PAD,), jnp.float32),
    ] + [pltpu.SemaphoreType.DMA] * (2 * DEG_RING),
)
def _sc_degrees(edge_hbm, zeros_hbm, out_hbm, ebuf, ones_v, dego_sh, degi_sh,
                *sems):
    c = lax.axis_index("c")
    s = lax.axis_index("s")
    w = c * NS + s

    esem = sems[0:DEG_RING]
    ssem = sems[DEG_RING:2 * DEG_RING]

    ones16 = jnp.ones((16,), jnp.float32)
    for j in range(ECHUNK // 16):
        ones_v[pl.ds(16 * j, 16)] = ones16

    pltpu.sync_copy(zeros_hbm, dego_sh.at[pl.ds(s * ELEMS_PER_TILE, ELEMS_PER_TILE)])
    pltpu.sync_copy(zeros_hbm, degi_sh.at[pl.ds(s * ELEMS_PER_TILE, ELEMS_PER_TILE)])

    plsc.subcore_barrier()

    def chunk_of(j):
        return w + NW * j

    def issue_in(b, j):
        k = chunk_of(j)
        pltpu.async_copy(edge_hbm.at[:, pl.ds(k * ECHUNK, ECHUNK)], ebuf.at[b],
                         esem[b])

    def wait_in(b, j):
        k = chunk_of(j)
        pltpu.make_async_copy(edge_hbm.at[:, pl.ds(k * ECHUNK, ECHUNK)],
                              ebuf.at[b], esem[b]).wait()

    def issue_work(b, j):
        pltpu.async_copy(ones_v, dego_sh.at[ebuf.at[b, 0]], ssem[b], add=True)
        pltpu.async_copy(ones_v, degi_sh.at[ebuf.at[b, 1]], ssem[b], add=True)

    def drain_work(b, j):
        pltpu.make_async_copy(ones_v, dego_sh.at[ebuf.at[b, 0]], ssem[b]).wait()
        pltpu.make_async_copy(ones_v, degi_sh.at[ebuf.at[b, 1]], ssem[b]).wait()

    issue_in(0, 0)
    issue_in(1, 1)

    def valid(j):
        return chunk_of(j) < TOT_ECHUNKS

    @pl.loop(0, 80, step=DEG_RING)
    def _main(g):
        for bi in range(DEG_RING):
            j = g + bi
            b = bi
            b2 = (bi + 2) % DEG_RING

            @pl.when(jnp.logical_and(j >= 2, valid(j - 2)))
            def _():
                drain_work(b2, j - 2)

            @pl.when(valid(j + 2))
            def _():
                issue_in(b2, j + 2)

            @pl.when(valid(j))
            def _():
                wait_in(b, j)
                issue_work(b, j)

    @pl.when(valid(78))
    def _():
        drain_work(78 % DEG_RING, 78)

    plsc.subcore_barrier()

    off = s * ELEMS_PER_TILE
    pltpu.sync_copy(dego_sh.at[pl.ds(off, ELEMS_PER_TILE)],
                    out_hbm.at[pl.ds((c * 2 + 0) * N_PAD + off, ELEMS_PER_TILE)])
    pltpu.sync_copy(degi_sh.at[pl.ds(off, ELEMS_PER_TILE)],
                    out_hbm.at[pl.ds((c * 2 + 1) * N_PAD + off, ELEMS_PER_TILE)])


SP_ER = 6
SP_RR = 3


@functools.partial(
    pl.kernel,
    out_type=jax.ShapeDtypeStruct((NC, N_NODES, D), jnp.float32),
    mesh=_sc_mesh,
    scratch_types=[
        pltpu.VMEM((SP_ER, 2, ECHUNK), jnp.int32),
        pltpu.VMEM((SP_RR, ECHUNK, D), jnp.float32),
        pltpu.VMEM_SHARED((N_NODES, D), jnp.float32),
    ] + [pltpu.SemaphoreType.DMA] * (SP_ER + SP_RR),
)
def _sc_spmm(h_hbm, edge_hbm, zrows_hbm, out_hbm, ebuf, rows, agg_sh, *sems):
    c = lax.axis_index("c")
    s = lax.axis_index("s")
    w = c * NS + s

    esem = sems[:SP_ER]
    gsem = sems[SP_ER:]

    @pl.when(s < 10)
    def _zero():
        pltpu.sync_copy(zrows_hbm, agg_sh.at[pl.ds(s * 1000, 1000)])

    plsc.subcore_barrier()

    def chunk_of(j):
        return w + NW * j

    def valid(j):
        return chunk_of(j) < TOT_ECHUNKS

    def issue_in(e, j):
        pltpu.async_copy(edge_hbm.at[:, pl.ds(chunk_of(j) * ECHUNK, ECHUNK)],
                         ebuf.at[e], esem[e])

    def wait_in(e, j):
        pltpu.make_async_copy(edge_hbm.at[:, pl.ds(chunk_of(j) * ECHUNK, ECHUNK)],
                              ebuf.at[e], esem[e]).wait()

    def issue_gather(r, e):
        pltpu.async_copy(h_hbm.at[ebuf.at[e, 0]], rows.at[r], gsem[r])

    def wait_gather(r, e):
        pltpu.make_async_copy(h_hbm.at[ebuf.at[e, 0]], rows.at[r], gsem[r]).wait()

    issue_in(0, 0)
    issue_in(1, 1)
    issue_in(2, 2)
    wait_in(0, 0)
    issue_gather(0, 0)
    wait_in(1, 1)
    issue_gather(1, 1)

    @pl.loop(0, 84, step=SP_ER)
    def _main(g):
        for bi in range(SP_ER):
            j = g + bi
            e0 = bi
            e2 = (bi + 2) % SP_ER
            e3 = (bi + 3) % SP_ER
            r0 = bi % SP_RR
            r2 = (bi + 2) % SP_RR

            @pl.when(valid(j + 2))
            def _():
                wait_in(e2, j + 2)
                issue_gather(r2, e2)

            @pl.when(valid(j + 3))
            def _():
                issue_in(e3, j + 3)

            @pl.when(valid(j))
            def _():
                wait_gather(r0, e0)
                pltpu.sync_copy(rows.at[r0], agg_sh.at[ebuf.at[e0, 1]], add=True)

    plsc.subcore_barrier()

    @pl.when(s < 10)
    def _dump():
        pltpu.sync_copy(agg_sh.at[pl.ds(s * 1000, 1000)],
                        out_hbm.at[c, pl.ds(s * 1000, 1000)])


def _prep_body(x_ref, dp_ref, hs_ref, ro_ref, ri_ref):
    do = dp_ref[pl.ds(0, N_PAD)] + dp_ref[pl.ds(2 * N_PAD, N_PAD)]
    di = dp_ref[pl.ds(N_PAD, N_PAD)] + dp_ref[pl.ds(3 * N_PAD, N_PAD)]
    ro = lax.rsqrt(jnp.maximum(do, 1.0)).reshape(N_PAD, 1)[:N_NODES]
    ri = lax.rsqrt(jnp.maximum(di, 1.0)).reshape(N_PAD, 1)[:N_NODES]
    ro_ref[...] = ro
    ri_ref[...] = ri
    hs_ref[...] = x_ref[...] * ro


def _tc_prep(x, degp):
    return pl.pallas_call(
        _prep_body,
        out_shape=[
            jax.ShapeDtypeStruct((N_NODES, D), jnp.float32),
            jax.ShapeDtypeStruct((N_NODES, 1), jnp.float32),
            jax.ShapeDtypeStruct((N_NODES, 1), jnp.float32),
        ],
    )(x, degp)


ROW_BLK = 1000


def _layer_body(agg_ref, ri_ref, ro_ref, w_ref, b_ref, out_ref):
    a = (agg_ref[0] + agg_ref[1]) * ri_ref[...]
    y = jnp.dot(a, w_ref[...], preferred_element_type=jnp.float32) + b_ref[...]
    out_ref[...] = jnp.maximum(y, 0.0) * ro_ref[...]


def _tc_layer1(aggp, ri, ro, w, b):
    grid = (N_NODES // ROW_BLK,)
    return pl.pallas_call(
        _layer_body,
        grid=grid,
        in_specs=[
            pl.BlockSpec((NC, ROW_BLK, D), lambda i: (0, i, 0)),
            pl.BlockSpec((ROW_BLK, 1), lambda i: (i, 0)),
            pl.BlockSpec((ROW_BLK, 1), lambda i: (i, 0)),
            pl.BlockSpec((D, D), lambda i: (0, 0)),
            pl.BlockSpec((1, D), lambda i: (0, 0)),
        ],
        out_specs=pl.BlockSpec((ROW_BLK, D), lambda i: (i, 0)),
        out_shape=jax.ShapeDtypeStruct((N_NODES, D), jnp.float32),
    )(aggp, ri, ro, w, b)


def _final_body(agg_ref, ri_ref, w2_ref, b2_ref, wf1_ref, bf1_ref,
                wf3_ref, bf3_ref, out_ref):
    a = (agg_ref[0] + agg_ref[1]) * ri_ref[...]
    h2 = jnp.maximum(
        jnp.dot(a, w2_ref[...], preferred_element_type=jnp.float32) + b2_ref[...], 0.0)
    h3 = jnp.maximum(
        jnp.dot(h2, wf1_ref[...], preferred_element_type=jnp.float32) + bf1_ref[...], 0.0)
    out_ref[...] = (
        jnp.dot(h3, wf3_ref[...], preferred_element_type=jnp.float32) + bf3_ref[...])


def _tc_final(aggp, ri, w2, b2, wf1, bf1, wf3, bf3):
    grid = (N_NODES // ROW_BLK,)
    wspec = pl.BlockSpec((D, D), lambda i: (0, 0))
    bspec = pl.BlockSpec((1, D), lambda i: (0, 0))
    return pl.pallas_call(
        _final_body,
        grid=grid,
        in_specs=[
            pl.BlockSpec((NC, ROW_BLK, D), lambda i: (0, i, 0)),
            pl.BlockSpec((ROW_BLK, 1), lambda i: (i, 0)),
            wspec, bspec, wspec, bspec, wspec, bspec,
        ],
        out_specs=pl.BlockSpec((ROW_BLK, D), lambda i: (i, 0)),
        out_shape=jax.ShapeDtypeStruct((N_NODES, D), jnp.float32),
    )(aggp, ri, w2, b2, wf1, bf1, wf3, bf3)


def kernel(x, edge_index, W1, b1, W2, b2, Wf1, bf1, Wf3, bf3):
    ei = edge_index.astype(jnp.int32)

    zeros1d = jnp.zeros((ELEMS_PER_TILE,), jnp.float32)
    zrows = jnp.zeros((1000, D), jnp.float32)

    degp = _sc_degrees(ei, zeros1d)

    hs0, ro, ri = _tc_prep(x, degp)

    agg1 = _sc_spmm(hs0, ei, zrows)
    h1s = _tc_layer1(agg1, ri, ro, W1, b1.reshape(1, D))

    agg2 = _sc_spmm(h1s, ei, zrows)
    z = _tc_final(agg2, ri, W2, b2.reshape(1, D),
                  Wf1, bf1.reshape(1, D), Wf3, bf3.reshape(1, D))
    return z


# speedup vs baseline: 15.2985x; 1.0013x over previous
import functools

import jax
import jax.numpy as jnp
from jax import lax
from jax.experimental import pallas as pl
from jax.experimental.pallas import tpu as pltpu
from jax.experimental.pallas import tpu_sc as plsc

N_NODES = 10000
N_EDGES = 320000
D = 128

NC = 2
NS = 16
NW = NC * NS
EDGES_PER_TILE = N_EDGES // NW
CHUNK = 80
CHUNKS_PER_TILE = EDGES_PER_TILE // CHUNK
ROWS_PER_TILE = N_NODES // NS

_sc_mesh = plsc.VectorSubcoreMesh(core_axis_name="c", subcore_axis_name="s")


N_PAD = 10240
ELEMS_PER_TILE = N_PAD // NS
ECHUNK = 128
TOT_ECHUNKS = N_EDGES // ECHUNK
DEG_RING = 6


@functools.partial(
    pl.kernel,
    out_type=jax.ShapeDtypeStruct((NC * 2 * N_PAD,), jnp.float32),
    mesh=_sc_mesh,
    scratch_types=[
        pltpu.VMEM((DEG_RING, 2, ECHUNK), jnp.int32),
        pltpu.VMEM((ECHUNK,), jnp.float32),
        pltpu.VMEM_SHARED((N_PAD,), jnp.float32),
        pltpu.VMEM_SHARED((N_PAD,), jnp.float32),
    ] + [pltpu.SemaphoreType.DMA] * (2 * DEG_RING),
)
def _sc_degrees(edge_hbm, zeros_hbm, out_hbm, ebuf, ones_v, dego_sh, degi_sh,
                *sems):
    c = lax.axis_index("c")
    s = lax.axis_index("s")
    w = c * NS + s

    esem = sems[0:DEG_RING]
    ssem = sems[DEG_RING:2 * DEG_RING]

    ones16 = jnp.ones((16,), jnp.float32)
    for j in range(ECHUNK // 16):
        ones_v[pl.ds(16 * j, 16)] = ones16

    pltpu.sync_copy(zeros_hbm, dego_sh.at[pl.ds(s * ELEMS_PER_TILE, ELEMS_PER_TILE)])
    pltpu.sync_copy(zeros_hbm, degi_sh.at[pl.ds(s * ELEMS_PER_TILE, ELEMS_PER_TILE)])

    plsc.subcore_barrier()

    def chunk_of(j):
        return w + NW * j

    def issue_in(b, j):
        k = chunk_of(j)
        pltpu.async_copy(edge_hbm.at[:, pl.ds(k * ECHUNK, ECHUNK)], ebuf.at[b],
                         esem[b])

    def wait_in(b, j):
        k = chunk_of(j)
        pltpu.make_async_copy(edge_hbm.at[:, pl.ds(k * ECHUNK, ECHUNK)],
                              ebuf.at[b], esem[b]).wait()

    def issue_work(b, j):
        pltpu.async_copy(ones_v, dego_sh.at[ebuf.at[b, 0]], ssem[b], add=True)
        pltpu.async_copy(ones_v, degi_sh.at[ebuf.at[b, 1]], ssem[b], add=True)

    def drain_work(b, j):
        pltpu.make_async_copy(ones_v, dego_sh.at[ebuf.at[b, 0]], ssem[b]).wait()
        pltpu.make_async_copy(ones_v, degi_sh.at[ebuf.at[b, 1]], ssem[b]).wait()

    issue_in(0, 0)
    issue_in(1, 1)

    def valid(j):
        return chunk_of(j) < TOT_ECHUNKS

    @pl.loop(0, 84, step=DEG_RING)
    def _main(g):
        for bi in range(DEG_RING):
            j = g + bi
            b = bi
            b2 = (bi + 2) % DEG_RING

            @pl.when(jnp.logical_and(j >= 4, valid(j - 4)))
            def _():
                drain_work(b2, j - 4)

            @pl.when(valid(j + 2))
            def _():
                issue_in(b2, j + 2)

            @pl.when(valid(j))
            def _():
                wait_in(b, j)
                issue_work(b, j)

    plsc.subcore_barrier()

    off = s * ELEMS_PER_TILE
    pltpu.sync_copy(dego_sh.at[pl.ds(off, ELEMS_PER_TILE)],
                    out_hbm.at[pl.ds((c * 2 + 0) * N_PAD + off, ELEMS_PER_TILE)])
    pltpu.sync_copy(degi_sh.at[pl.ds(off, ELEMS_PER_TILE)],
                    out_hbm.at[pl.ds((c * 2 + 1) * N_PAD + off, ELEMS_PER_TILE)])


SP_ER = 6
SP_RR = 3


@functools.partial(
    pl.kernel,
    out_type=jax.ShapeDtypeStruct((NC, N_NODES, D), jnp.float32),
    mesh=_sc_mesh,
    scratch_types=[
        pltpu.VMEM((SP_ER, 2, ECHUNK), jnp.int32),
        pltpu.VMEM((SP_RR, ECHUNK, D), jnp.float32),
        pltpu.VMEM_SHARED((N_NODES, D), jnp.float32),
    ] + [pltpu.SemaphoreType.DMA] * (SP_ER + 2 * SP_RR),
)
def _sc_spmm(h_hbm, edge_hbm, zrows_hbm, out_hbm, ebuf, rows, agg_sh, *sems):
    c = lax.axis_index("c")
    s = lax.axis_index("s")
    w = c * NS + s

    esem = sems[:SP_ER]
    gsem = sems[SP_ER:SP_ER + SP_RR]
    ssem = sems[SP_ER + SP_RR:]

    @pl.when(s < 10)
    def _zero():
        pltpu.sync_copy(zrows_hbm, agg_sh.at[pl.ds(s * 1000, 1000)])

    plsc.subcore_barrier()

    def chunk_of(j):
        return w + NW * j

    def valid(j):
        return chunk_of(j) < TOT_ECHUNKS

    def issue_in(e, j):
        pltpu.async_copy(edge_hbm.at[:, pl.ds(chunk_of(j) * ECHUNK, ECHUNK)],
                         ebuf.at[e], esem[e])

    def wait_in(e, j):
        pltpu.make_async_copy(edge_hbm.at[:, pl.ds(chunk_of(j) * ECHUNK, ECHUNK)],
                              ebuf.at[e], esem[e]).wait()

    def issue_gather(r, e):
        pltpu.async_copy(h_hbm.at[ebuf.at[e, 0]], rows.at[r], gsem[r])

    def wait_gather(r, e):
        pltpu.make_async_copy(h_hbm.at[ebuf.at[e, 0]], rows.at[r], gsem[r]).wait()

    def issue_scatter(r, e):
        pltpu.async_copy(rows.at[r], agg_sh.at[ebuf.at[e, 1]], ssem[r], add=True)

    def drain_scatter(r, e):
        pltpu.make_async_copy(rows.at[r], agg_sh.at[ebuf.at[e, 1]], ssem[r]).wait()

    issue_in(0, 0)
    issue_in(1, 1)
    issue_in(2, 2)
    wait_in(0, 0)
    issue_gather(0, 0)
    wait_in(1, 1)
    issue_gather(1, 1)

    @pl.loop(0, 84, step=SP_ER)
    def _main(g):
        for bi in range(SP_ER):
            j = g + bi
            e0 = bi
            em1 = (bi + SP_ER - 1) % SP_ER
            e2 = (bi + 2) % SP_ER
            e3 = (bi + 3) % SP_ER
            r0 = bi % SP_RR
            rm1 = (bi + SP_RR - 1) % SP_RR
            r2 = (bi + 2) % SP_RR

            @pl.when(jnp.logical_and(j >= 1, valid(j - 1)))
            def _():
                drain_scatter(rm1, em1)

            @pl.when(valid(j + 2))
            def _():
                wait_in(e2, j + 2)
                issue_gather(r2, e2)

            @pl.when(valid(j + 3))
            def _():
                issue_in(e3, j + 3)

            @pl.when(valid(j))
            def _():
                wait_gather(r0, e0)
                issue_scatter(r0, e0)

    plsc.subcore_barrier()

    @pl.when(s < 10)
    def _dump():
        pltpu.sync_copy(agg_sh.at[pl.ds(s * 1000, 1000)],
                        out_hbm.at[c, pl.ds(s * 1000, 1000)])


def _prep_body(x_ref, dp_ref, hs_ref, ro_ref, ri_ref):
    do = dp_ref[pl.ds(0, N_PAD)] + dp_ref[pl.ds(2 * N_PAD, N_PAD)]
    di = dp_ref[pl.ds(N_PAD, N_PAD)] + dp_ref[pl.ds(3 * N_PAD, N_PAD)]
    ro = lax.rsqrt(jnp.maximum(do, 1.0)).reshape(N_PAD, 1)[:N_NODES]
    ri = lax.rsqrt(jnp.maximum(di, 1.0)).reshape(N_PAD, 1)[:N_NODES]
    ro_ref[...] = ro
    ri_ref[...] = ri
    hs_ref[...] = x_ref[...] * ro


def _tc_prep(x, degp):
    return pl.pallas_call(
        _prep_body,
        out_shape=[
            jax.ShapeDtypeStruct((N_NODES, D), jnp.float32),
            jax.ShapeDtypeStruct((N_NODES, 1), jnp.float32),
            jax.ShapeDtypeStruct((N_NODES, 1), jnp.float32),
        ],
    )(x, degp)


ROW_BLK = 1000


def _layer_body(agg_ref, ri_ref, ro_ref, w_ref, b_ref, out_ref):
    a = (agg_ref[0] + agg_ref[1]) * ri_ref[...]
    y = jnp.dot(a, w_ref[...], preferred_element_type=jnp.float32) + b_ref[...]
    out_ref[...] = jnp.maximum(y, 0.0) * ro_ref[...]


def _tc_layer1(aggp, ri, ro, w, b):
    grid = (N_NODES // ROW_BLK,)
    return pl.pallas_call(
        _layer_body,
        grid=grid,
        in_specs=[
            pl.BlockSpec((NC, ROW_BLK, D), lambda i: (0, i, 0)),
            pl.BlockSpec((ROW_BLK, 1), lambda i: (i, 0)),
            pl.BlockSpec((ROW_BLK, 1), lambda i: (i, 0)),
            pl.BlockSpec((D, D), lambda i: (0, 0)),
            pl.BlockSpec((1, D), lambda i: (0, 0)),
        ],
        out_specs=pl.BlockSpec((ROW_BLK, D), lambda i: (i, 0)),
        out_shape=jax.ShapeDtypeStruct((N_NODES, D), jnp.float32),
    )(aggp, ri, ro, w, b)


def _final_body(agg_ref, ri_ref, w2_ref, b2_ref, wf1_ref, bf1_ref,
                wf3_ref, bf3_ref, out_ref):
    a = (agg_ref[0] + agg_ref[1]) * ri_ref[...]
    h2 = jnp.maximum(
        jnp.dot(a, w2_ref[...], preferred_element_type=jnp.float32) + b2_ref[...], 0.0)
    h3 = jnp.maximum(
        jnp.dot(h2, wf1_ref[...], preferred_element_type=jnp.float32) + bf1_ref[...], 0.0)
    out_ref[...] = (
        jnp.dot(h3, wf3_ref[...], preferred_element_type=jnp.float32) + bf3_ref[...])


def _tc_final(aggp, ri, w2, b2, wf1, bf1, wf3, bf3):
    grid = (N_NODES // ROW_BLK,)
    wspec = pl.BlockSpec((D, D), lambda i: (0, 0))
    bspec = pl.BlockSpec((1, D), lambda i: (0, 0))
    return pl.pallas_call(
        _final_body,
        grid=grid,
        in_specs=[
            pl.BlockSpec((NC, ROW_BLK, D), lambda i: (0, i, 0)),
            pl.BlockSpec((ROW_BLK, 1), lambda i: (i, 0)),
            wspec, bspec, wspec, bspec, wspec, bspec,
        ],
        out_specs=pl.BlockSpec((ROW_BLK, D), lambda i: (i, 0)),
        out_shape=jax.ShapeDtypeStruct((N_NODES, D), jnp.float32),
    )(aggp, ri, w2, b2, wf1, bf1, wf3, bf3)


def kernel(x, edge_index, W1, b1, W2, b2, Wf1, bf1, Wf3, bf3):
    ei = edge_index.astype(jnp.int32)

    zeros1d = jnp.zeros((ELEMS_PER_TILE,), jnp.float32)
    zrows = jnp.zeros((1000, D), jnp.float32)

    degp = _sc_degrees(ei, zeros1d)

    hs0, ro, ri = _tc_prep(x, degp)

    agg1 = _sc_spmm(hs0, ei, zrows)
    h1s = _tc_layer1(agg1, ri, ro, W1, b1.reshape(1, D))

    agg2 = _sc_spmm(h1s, ei, zrows)
    z = _tc_final(agg2, ri, W2, b2.reshape(1, D),
                  Wf1, bf1.reshape(1, D), Wf3, bf3.reshape(1, D))
    return z
